# baseline (speedup 1.0000x reference)
import functools

import jax
import jax.numpy as jnp
from jax import lax
from jax.experimental import pallas as pl
from jax.experimental.pallas import tpu as pltpu
from jax.experimental.pallas import tpu_sc as plsc

B = 16384
D = 128
EPS = 1e-5


try:
    _info = plsc.get_sparse_core_info()
    _NC = _info.num_cores
    _NS = _info.num_subcores
except Exception:
    _NC, _NS = 2, 16
_NW = _NC * _NS
_BPW = B // _NW
_CH = 64
_NCH = _BPW // _CH


def _sc_gather4(uidx, iidx, t0, t1, t2, t3):
    mesh = plsc.VectorSubcoreMesh(core_axis_name="c", subcore_axis_name="s")
    f32 = jnp.float32

    @functools.partial(
        pl.kernel,
        mesh=mesh,
        out_type=[jax.ShapeDtypeStruct((B, D), f32) for _ in range(4)],
        scratch_types=(
            [pltpu.VMEM((_CH,), jnp.int32) for _ in range(4)]
            + [pltpu.VMEM((_CH, D), f32) for _ in range(8)]
            + [pltpu.SemaphoreType.DMA for _ in range(4)]
        ),
    )
    def gather_k(uidx_h, iidx_h, t0_h, t1_h, t2_h, t3_h,
                 o0_h, o1_h, o2_h, o3_h,
                 uv0, uv1, iv0, iv1,
                 b00, b10, b20, b30, b01, b11, b21, b31,
                 g0, g1, s0, s1):
        uv = (uv0, uv1)
        iv = (iv0, iv1)
        bufs = ((b00, b10, b20, b30), (b01, b11, b21, b31))
        gsem = (g0, g1)
        ssem = (s0, s1)
        tabs = (t0_h, t1_h, t2_h, t3_h)
        outs = (o0_h, o1_h, o2_h, o3_h)
        wid = lax.axis_index("s") * _NC + lax.axis_index("c")
        base = wid * _BPW

        gh = [None] * _NCH
        sh = [None] * _NCH
        pltpu.sync_copy(uidx_h.at[pl.ds(base, _CH)], uv[0])
        pltpu.sync_copy(iidx_h.at[pl.ds(base, _CH)], iv[0])
        for c in range(_NCH):
            p = c % 2
            if c >= 2:
                for h in sh[c - 2]:
                    h.wait()
            idxs = (uv[p], iv[p], uv[p], iv[p])
            gh[c] = tuple(
                pltpu.async_copy(tabs[t].at[idxs[t]], bufs[p][t], gsem[p])
                for t in range(4))
            if c + 1 < _NCH:
                off_n = base + (c + 1) * _CH
                pltpu.sync_copy(uidx_h.at[pl.ds(off_n, _CH)], uv[1 - p])
                pltpu.sync_copy(iidx_h.at[pl.ds(off_n, _CH)], iv[1 - p])
            if c >= 1:
                q = 1 - p
                off_p = base + (c - 1) * _CH
                for h in gh[c - 1]:
                    h.wait()
                sh[c - 1] = tuple(
                    pltpu.async_copy(bufs[q][t],
                                     outs[t].at[pl.ds(off_p, _CH)], ssem[q])
                    for t in range(4))
        c = _NCH - 1
        p = c % 2
        for h in gh[c]:
            h.wait()
        off_p = base + c * _CH
        sh[c] = tuple(
            pltpu.async_copy(bufs[p][t], outs[t].at[pl.ds(off_p, _CH)],
                             ssem[p])
            for t in range(4))
        for h in sh[c - 1]:
            h.wait()
        for h in sh[c]:
            h.wait()

    return gather_k(uidx, iidx, t0, t1, t2, t3)


_BLK = 2048
_NB = B // _BLK


def _leaky(z):
    return jnp.where(z > 0, z, 0.1 * z)


def _accum_stats(a, st_ref):
    ps = jnp.stack([jnp.sum(a, axis=0), jnp.sum(a * a, axis=0)])

    @pl.when(pl.program_id(0) == 0)
    def _():
        st_ref[...] = ps

    @pl.when(pl.program_id(0) > 0)
    def _():
        st_ref[...] = st_ref[...] + ps


def _norm_params(st, g, be):
    m = st[0] * (1.0 / B)
    var = st[1] * (1.0 / B) - m * m
    scale = g * lax.rsqrt(var + EPS)
    shift = be - m * scale
    return scale, shift


def _stage1_body(um_ref, im_ref, w_ref, b_ref, h_ref, st_ref):
    w = w_ref[...]
    z = (jnp.dot(um_ref[...], w[:D], preferred_element_type=jnp.float32)
         + jnp.dot(im_ref[...], w[D:], preferred_element_type=jnp.float32)
         + b_ref[...])
    a = _leaky(z)
    h_ref[...] = a
    _accum_stats(a, st_ref)


def _stageN_body(h_in_ref, st_in_ref, g_ref, be_ref, w_ref, b_ref,
                 h_ref, st_ref):
    scale, shift = _norm_params(st_in_ref[...], g_ref[...], be_ref[...])
    x = h_in_ref[...] * scale + shift
    z = jnp.dot(x, w_ref[...], preferred_element_type=jnp.float32) + b_ref[...]
    a = _leaky(z)
    h_ref[...] = a
    _accum_stats(a, st_ref)


def _final_body(h3_ref, st_ref, g_ref, be_ref, ug_ref, ig_ref,
                wo_ref, bo_ref, o_ref):
    scale, shift = _norm_params(st_ref[...], g_ref[...], be_ref[...])
    z3 = h3_ref[...] * scale + shift
    gmf = ug_ref[...] * ig_ref[...]
    wo = wo_ref[...][:, 0]
    s = (jnp.sum(gmf * wo[:D] + z3 * wo[D:], axis=1) + bo_ref[0])
    o_ref[...] = jax.nn.sigmoid(s)


def _full_spec(ndim):
    return pl.BlockSpec(None, lambda i: (0,) * ndim)


def _row_spec(h):
    return pl.BlockSpec((_BLK, h), lambda i: (i, 0))


def kernel(user_indices, item_indices, ue_gmf, ie_gmf, ue_mlp, ie_mlp,
           W1, b1, g1, be1, W2, b2, g2, be2, W3, b3, g3, be3, Wo, bo):
    uidx = user_indices.astype(jnp.int32)
    iidx = item_indices.astype(jnp.int32)

    um, im, ug, ig = _sc_gather4(uidx, iidx, ue_mlp, ie_mlp, ue_gmf, ie_gmf)

    f32 = jnp.float32
    h1, st1 = pl.pallas_call(
        _stage1_body,
        grid=(_NB,),
        in_specs=[_row_spec(D), _row_spec(D), _full_spec(2), _full_spec(1)],
        out_specs=[_row_spec(512), _full_spec(2)],
        out_shape=[jax.ShapeDtypeStruct((B, 512), f32),
                   jax.ShapeDtypeStruct((2, 512), f32)],
    )(um, im, W1, b1)

    h2, st2 = pl.pallas_call(
        _stageN_body,
        grid=(_NB,),
        in_specs=[_row_spec(512), _full_spec(2), _full_spec(1), _full_spec(1),
                  _full_spec(2), _full_spec(1)],
        out_specs=[_row_spec(256), _full_spec(2)],
        out_shape=[jax.ShapeDtypeStruct((B, 256), f32),
                   jax.ShapeDtypeStruct((2, 256), f32)],
    )(h1, st1, g1, be1, W2, b2)

    h3, st3 = pl.pallas_call(
        _stageN_body,
        grid=(_NB,),
        in_specs=[_row_spec(256), _full_spec(2), _full_spec(1), _full_spec(1),
                  _full_spec(2), _full_spec(1)],
        out_specs=[_row_spec(128), _full_spec(2)],
        out_shape=[jax.ShapeDtypeStruct((B, 128), f32),
                   jax.ShapeDtypeStruct((2, 128), f32)],
    )(h2, st2, g2, be2, W3, b3)

    out = pl.pallas_call(
        _final_body,
        grid=(_NB,),
        in_specs=[_row_spec(128), _full_spec(2), _full_spec(1), _full_spec(1),
                  _row_spec(D), _row_spec(D), _full_spec(2), _full_spec(1)],
        out_specs=pl.BlockSpec((_BLK,), lambda i: (i,)),
        out_shape=jax.ShapeDtypeStruct((B,), f32),
    )(h3, st3, g3, be3, ug, ig, Wo, bo)

    return out


# speedup vs baseline: 1.0102x; 1.0102x over previous
import functools

import jax
import jax.numpy as jnp
from jax import lax
from jax.experimental import pallas as pl
from jax.experimental.pallas import tpu as pltpu
from jax.experimental.pallas import tpu_sc as plsc

B = 16384
D = 128
EPS = 1e-5


try:
    _info = plsc.get_sparse_core_info()
    _NC = _info.num_cores
    _NS = _info.num_subcores
except Exception:
    _NC, _NS = 2, 16
_NW = _NC * _NS
_BPW = B // _NW
_CH = 128
_NCH = _BPW // _CH


def _sc_gather4(uidx, iidx, t0, t1, t2, t3):
    mesh = plsc.VectorSubcoreMesh(core_axis_name="c", subcore_axis_name="s")
    f32 = jnp.float32

    @functools.partial(
        pl.kernel,
        mesh=mesh,
        out_type=[jax.ShapeDtypeStruct((B, D), f32) for _ in range(4)],
        scratch_types=(
            [pltpu.VMEM((_CH,), jnp.int32) for _ in range(4)]
            + [pltpu.VMEM((_CH, D), f32) for _ in range(4)]
            + [pltpu.SemaphoreType.DMA for _ in range(4)]
        ),
    )
    def gather_k(uidx_h, iidx_h, t0_h, t1_h, t2_h, t3_h,
                 o0_h, o1_h, o2_h, o3_h,
                 uv0, uv1, iv0, iv1,
                 b00, b10, b01, b11,
                 g0, g1, s0, s1):
        uv = (uv0, uv1)
        iv = (iv0, iv1)
        ubuf = (b00, b01)
        ibuf = (b10, b11)
        gsem = (g0, g1)
        ssem = (s0, s1)
        wid = lax.axis_index("s") * _NC + lax.axis_index("c")
        base = wid * _BPW

        jobs = [(t0_h, t1_h, o0_h, o1_h, c) for c in range(_NCH)]
        jobs += [(t2_h, t3_h, o2_h, o3_h, c) for c in range(_NCH)]
        nj = len(jobs)

        gh = [None] * nj
        sh = [None] * nj
        pltpu.sync_copy(uidx_h.at[pl.ds(base + jobs[0][4] * _CH, _CH)], uv[0])
        pltpu.sync_copy(iidx_h.at[pl.ds(base + jobs[0][4] * _CH, _CH)], iv[0])
        for j in range(nj):
            p = j % 2
            tu, ti, ou, oi, c = jobs[j]
            if j >= 2:
                sh[j - 2][0].wait()
                sh[j - 2][1].wait()
            gh[j] = (pltpu.async_copy(tu.at[uv[p]], ubuf[p], gsem[p]),
                     pltpu.async_copy(ti.at[iv[p]], ibuf[p], gsem[p]))
            if j >= 1:
                q = 1 - p
                tu_p, ti_p, ou_p, oi_p, c_p = jobs[j - 1]
                off_p = base + c_p * _CH
                gh[j - 1][0].wait()
                gh[j - 1][1].wait()
                sh[j - 1] = (
                    pltpu.async_copy(ubuf[q], ou_p.at[pl.ds(off_p, _CH)],
                                     ssem[q]),
                    pltpu.async_copy(ibuf[q], oi_p.at[pl.ds(off_p, _CH)],
                                     ssem[q]),
                )
            if j + 1 < nj:
                off_n = base + jobs[j + 1][4] * _CH
                pltpu.sync_copy(uidx_h.at[pl.ds(off_n, _CH)], uv[1 - p])
                pltpu.sync_copy(iidx_h.at[pl.ds(off_n, _CH)], iv[1 - p])
        j = nj - 1
        p = j % 2
        gh[j][0].wait()
        gh[j][1].wait()
        tu_p, ti_p, ou_p, oi_p, c_p = jobs[j]
        off_p = base + c_p * _CH
        sh[j] = (pltpu.async_copy(ubuf[p], ou_p.at[pl.ds(off_p, _CH)],
                                  ssem[p]),
                 pltpu.async_copy(ibuf[p], oi_p.at[pl.ds(off_p, _CH)],
                                  ssem[p]))
        sh[j - 1][0].wait()
        sh[j - 1][1].wait()
        sh[j][0].wait()
        sh[j][1].wait()

    return gather_k(uidx, iidx, t0, t1, t2, t3)


_BLK = 2048
_NB = B // _BLK


def _leaky(z):
    return jnp.where(z > 0, z, 0.1 * z)


def _accum_stats(a, st_ref):
    ps = jnp.stack([jnp.sum(a, axis=0), jnp.sum(a * a, axis=0)])

    @pl.when(pl.program_id(0) == 0)
    def _():
        st_ref[...] = ps

    @pl.when(pl.program_id(0) > 0)
    def _():
        st_ref[...] = st_ref[...] + ps


def _norm_params(st, g, be):
    m = st[0] * (1.0 / B)
    var = st[1] * (1.0 / B) - m * m
    scale = g * lax.rsqrt(var + EPS)
    shift = be - m * scale
    return scale, shift


def _stage1_body(um_ref, im_ref, w_ref, b_ref, h_ref, st_ref):
    w = w_ref[...]
    z = (jnp.dot(um_ref[...], w[:D], preferred_element_type=jnp.float32)
         + jnp.dot(im_ref[...], w[D:], preferred_element_type=jnp.float32)
         + b_ref[...])
    a = _leaky(z)
    h_ref[...] = a
    _accum_stats(a, st_ref)


def _stageN_body(h_in_ref, st_in_ref, g_ref, be_ref, w_ref, b_ref,
                 h_ref, st_ref):
    scale, shift = _norm_params(st_in_ref[...], g_ref[...], be_ref[...])
    x = h_in_ref[...] * scale + shift
    z = jnp.dot(x, w_ref[...], preferred_element_type=jnp.float32) + b_ref[...]
    a = _leaky(z)
    h_ref[...] = a
    _accum_stats(a, st_ref)


def _final_body(h3_ref, st_ref, g_ref, be_ref, ug_ref, ig_ref,
                wo_ref, bo_ref, o_ref):
    scale, shift = _norm_params(st_ref[...], g_ref[...], be_ref[...])
    z3 = h3_ref[...] * scale + shift
    gmf = ug_ref[...] * ig_ref[...]
    wo = wo_ref[...][:, 0]
    s = (jnp.sum(gmf * wo[:D] + z3 * wo[D:], axis=1) + bo_ref[0])
    o_ref[...] = jax.nn.sigmoid(s)


def _full_spec(ndim):
    return pl.BlockSpec(None, lambda i: (0,) * ndim)


def _row_spec(h):
    return pl.BlockSpec((_BLK, h), lambda i: (i, 0))


def kernel(user_indices, item_indices, ue_gmf, ie_gmf, ue_mlp, ie_mlp,
           W1, b1, g1, be1, W2, b2, g2, be2, W3, b3, g3, be3, Wo, bo):
    uidx = user_indices.astype(jnp.int32)
    iidx = item_indices.astype(jnp.int32)

    um, im, ug, ig = _sc_gather4(uidx, iidx, ue_mlp, ie_mlp, ue_gmf, ie_gmf)

    f32 = jnp.float32
    h1, st1 = pl.pallas_call(
        _stage1_body,
        grid=(_NB,),
        in_specs=[_row_spec(D), _row_spec(D), _full_spec(2), _full_spec(1)],
        out_specs=[_row_spec(512), _full_spec(2)],
        out_shape=[jax.ShapeDtypeStruct((B, 512), f32),
                   jax.ShapeDtypeStruct((2, 512), f32)],
    )(um, im, W1, b1)

    h2, st2 = pl.pallas_call(
        _stageN_body,
        grid=(_NB,),
        in_specs=[_row_spec(512), _full_spec(2), _full_spec(1), _full_spec(1),
                  _full_spec(2), _full_spec(1)],
        out_specs=[_row_spec(256), _full_spec(2)],
        out_shape=[jax.ShapeDtypeStruct((B, 256), f32),
                   jax.ShapeDtypeStruct((2, 256), f32)],
    )(h1, st1, g1, be1, W2, b2)

    h3, st3 = pl.pallas_call(
        _stageN_body,
        grid=(_NB,),
        in_specs=[_row_spec(256), _full_spec(2), _full_spec(1), _full_spec(1),
                  _full_spec(2), _full_spec(1)],
        out_specs=[_row_spec(128), _full_spec(2)],
        out_shape=[jax.ShapeDtypeStruct((B, 128), f32),
                   jax.ShapeDtypeStruct((2, 128), f32)],
    )(h2, st2, g2, be2, W3, b3)

    out = pl.pallas_call(
        _final_body,
        grid=(_NB,),
        in_specs=[_row_spec(128), _full_spec(2), _full_spec(1), _full_spec(1),
                  _row_spec(D), _row_spec(D), _full_spec(2), _full_spec(1)],
        out_specs=pl.BlockSpec((_BLK,), lambda i: (i,)),
        out_shape=jax.ShapeDtypeStruct((B,), f32),
    )(h3, st3, g3, be3, ug, ig, Wo, bo)

    return out


# speedup vs baseline: 1.1477x; 1.1361x over previous
import functools

import jax
import jax.numpy as jnp
from jax import lax
from jax.experimental import pallas as pl
from jax.experimental.pallas import tpu as pltpu
from jax.experimental.pallas import tpu_sc as plsc

B = 16384
D = 128
EPS = 1e-5


try:
    _info = plsc.get_sparse_core_info()
    _NC = _info.num_cores
    _NS = _info.num_subcores
except Exception:
    _NC, _NS = 2, 16
_NW = _NC * _NS
_BPW = B // _NW
_CH = 128
_NCH = _BPW // _CH


def _sc_gather2(uidx, iidx, t0, t1):
    mesh = plsc.VectorSubcoreMesh(core_axis_name="c", subcore_axis_name="s")
    f32 = jnp.float32

    @functools.partial(
        pl.kernel,
        mesh=mesh,
        out_type=[jax.ShapeDtypeStruct((B, D), f32) for _ in range(2)],
        scratch_types=(
            [pltpu.VMEM((_CH,), jnp.int32) for _ in range(4)]
            + [pltpu.VMEM((_CH, D), f32) for _ in range(4)]
            + [pltpu.SemaphoreType.DMA for _ in range(4)]
        ),
    )
    def gather_k(uidx_h, iidx_h, t0_h, t1_h,
                 o0_h, o1_h,
                 uv0, uv1, iv0, iv1,
                 b00, b10, b01, b11,
                 g0, g1, s0, s1):
        uv = (uv0, uv1)
        iv = (iv0, iv1)
        ubuf = (b00, b01)
        ibuf = (b10, b11)
        gsem = (g0, g1)
        ssem = (s0, s1)
        wid = lax.axis_index("s") * _NC + lax.axis_index("c")
        base = wid * _BPW

        jobs = [(t0_h, t1_h, o0_h, o1_h, c) for c in range(_NCH)]
        nj = len(jobs)

        gh = [None] * nj
        sh = [None] * nj
        pltpu.sync_copy(uidx_h.at[pl.ds(base + jobs[0][4] * _CH, _CH)], uv[0])
        pltpu.sync_copy(iidx_h.at[pl.ds(base + jobs[0][4] * _CH, _CH)], iv[0])
        for j in range(nj):
            p = j % 2
            tu, ti, ou, oi, c = jobs[j]
            if j >= 2:
                sh[j - 2][0].wait()
                sh[j - 2][1].wait()
            gh[j] = (pltpu.async_copy(tu.at[uv[p]], ubuf[p], gsem[p]),
                     pltpu.async_copy(ti.at[iv[p]], ibuf[p], gsem[p]))
            if j >= 1:
                q = 1 - p
                tu_p, ti_p, ou_p, oi_p, c_p = jobs[j - 1]
                off_p = base + c_p * _CH
                gh[j - 1][0].wait()
                gh[j - 1][1].wait()
                sh[j - 1] = (
                    pltpu.async_copy(ubuf[q], ou_p.at[pl.ds(off_p, _CH)],
                                     ssem[q]),
                    pltpu.async_copy(ibuf[q], oi_p.at[pl.ds(off_p, _CH)],
                                     ssem[q]),
                )
            if j + 1 < nj:
                off_n = base + jobs[j + 1][4] * _CH
                pltpu.sync_copy(uidx_h.at[pl.ds(off_n, _CH)], uv[1 - p])
                pltpu.sync_copy(iidx_h.at[pl.ds(off_n, _CH)], iv[1 - p])
        j = nj - 1
        p = j % 2
        gh[j][0].wait()
        gh[j][1].wait()
        tu_p, ti_p, ou_p, oi_p, c_p = jobs[j]
        off_p = base + c_p * _CH
        sh[j] = (pltpu.async_copy(ubuf[p], ou_p.at[pl.ds(off_p, _CH)],
                                  ssem[p]),
                 pltpu.async_copy(ibuf[p], oi_p.at[pl.ds(off_p, _CH)],
                                  ssem[p]))
        sh[j - 1][0].wait()
        sh[j - 1][1].wait()
        sh[j][0].wait()
        sh[j][1].wait()

    return gather_k(uidx, iidx, t0, t1)


_BLK = 2048
_NB = B // _BLK


def _leaky(z):
    return jnp.where(z > 0, z, 0.1 * z)


def _accum_stats(a, st_ref):
    ps = jnp.stack([jnp.sum(a, axis=0), jnp.sum(a * a, axis=0)])

    @pl.when(pl.program_id(0) == 0)
    def _():
        st_ref[...] = ps

    @pl.when(pl.program_id(0) > 0)
    def _():
        st_ref[...] = st_ref[...] + ps


def _norm_params(st, g, be):
    m = st[0] * (1.0 / B)
    var = st[1] * (1.0 / B) - m * m
    scale = g * lax.rsqrt(var + EPS)
    shift = be - m * scale
    return scale, shift


def _stage1_body(um_ref, im_ref, w_ref, b_ref, h_ref, st_ref):
    w = w_ref[...]
    z = (jnp.dot(um_ref[...].astype(jnp.bfloat16), w[:D],
                 preferred_element_type=jnp.float32)
         + jnp.dot(im_ref[...].astype(jnp.bfloat16), w[D:],
                   preferred_element_type=jnp.float32)
         + b_ref[...])
    a = _leaky(z)
    h_ref[...] = a.astype(jnp.bfloat16)
    _accum_stats(a, st_ref)


def _stageN_body(h_in_ref, st_in_ref, g_ref, be_ref, w_ref, b_ref,
                 h_ref, st_ref):
    scale, shift = _norm_params(st_in_ref[...], g_ref[...], be_ref[...])
    x = (h_in_ref[...].astype(jnp.float32) * scale + shift
         ).astype(jnp.bfloat16)
    z = jnp.dot(x, w_ref[...], preferred_element_type=jnp.float32) + b_ref[...]
    a = _leaky(z)
    h_ref[...] = a.astype(jnp.bfloat16)
    _accum_stats(a, st_ref)


def _final_body(h3_ref, st_ref, g_ref, be_ref, ug_ref, ig_ref,
                wo_ref, bo_ref, o_ref):
    scale, shift = _norm_params(st_ref[...], g_ref[...], be_ref[...])
    z3 = h3_ref[...].astype(jnp.float32) * scale + shift
    gmf = ug_ref[...] * ig_ref[...]
    wo = wo_ref[...][:, 0]
    s = (jnp.sum(gmf * wo[:D] + z3 * wo[D:], axis=1) + bo_ref[0])
    o_ref[...] = jax.nn.sigmoid(s)


def _full_spec(ndim):
    return pl.BlockSpec(None, lambda i: (0,) * ndim)


def _row_spec(h):
    return pl.BlockSpec((_BLK, h), lambda i: (i, 0))


def kernel(user_indices, item_indices, ue_gmf, ie_gmf, ue_mlp, ie_mlp,
           W1, b1, g1, be1, W2, b2, g2, be2, W3, b3, g3, be3, Wo, bo):
    uidx = user_indices.astype(jnp.int32)
    iidx = item_indices.astype(jnp.int32)

    um, im = _sc_gather2(uidx, iidx, ue_mlp, ie_mlp)
    ug, ig = _sc_gather2(uidx, iidx, ue_gmf, ie_gmf)

    f32 = jnp.float32
    bf16 = jnp.bfloat16
    h1, st1 = pl.pallas_call(
        _stage1_body,
        grid=(_NB,),
        in_specs=[_row_spec(D), _row_spec(D), _full_spec(2), _full_spec(1)],
        out_specs=[_row_spec(512), _full_spec(2)],
        out_shape=[jax.ShapeDtypeStruct((B, 512), bf16),
                   jax.ShapeDtypeStruct((2, 512), f32)],
    )(um, im, W1.astype(bf16), b1)

    h2, st2 = pl.pallas_call(
        _stageN_body,
        grid=(_NB,),
        in_specs=[_row_spec(512), _full_spec(2), _full_spec(1), _full_spec(1),
                  _full_spec(2), _full_spec(1)],
        out_specs=[_row_spec(256), _full_spec(2)],
        out_shape=[jax.ShapeDtypeStruct((B, 256), bf16),
                   jax.ShapeDtypeStruct((2, 256), f32)],
    )(h1, st1, g1, be1, W2.astype(bf16), b2)

    h3, st3 = pl.pallas_call(
        _stageN_body,
        grid=(_NB,),
        in_specs=[_row_spec(256), _full_spec(2), _full_spec(1), _full_spec(1),
                  _full_spec(2), _full_spec(1)],
        out_specs=[_row_spec(128), _full_spec(2)],
        out_shape=[jax.ShapeDtypeStruct((B, 128), bf16),
                   jax.ShapeDtypeStruct((2, 128), f32)],
    )(h2, st2, g2, be2, W3.astype(bf16), b3)

    out = pl.pallas_call(
        _final_body,
        grid=(_NB,),
        in_specs=[_row_spec(128), _full_spec(2), _full_spec(1), _full_spec(1),
                  _row_spec(D), _row_spec(D), _full_spec(2), _full_spec(1)],
        out_specs=pl.BlockSpec((_BLK,), lambda i: (i,)),
        out_shape=jax.ShapeDtypeStruct((B,), f32),
    )(h3, st3, g3, be3, ug, ig, Wo, bo)

    return out


# speedup vs baseline: 1.1738x; 1.0227x over previous
import functools

import jax
import jax.numpy as jnp
from jax import lax
from jax.experimental import pallas as pl
from jax.experimental.pallas import tpu as pltpu
from jax.experimental.pallas import tpu_sc as plsc

B = 16384
D = 128
EPS = 1e-5


try:
    _info = plsc.get_sparse_core_info()
    _NC = _info.num_cores
    _NS = _info.num_subcores
except Exception:
    _NC, _NS = 2, 16
_NW = _NC * _NS
_BPW = B // _NW
_CH = 128
_NCH = _BPW // _CH


def _sc_gather2(uidx, iidx, t0, t1):
    mesh = plsc.VectorSubcoreMesh(core_axis_name="c", subcore_axis_name="s")
    f32 = jnp.float32

    @functools.partial(
        pl.kernel,
        mesh=mesh,
        out_type=[jax.ShapeDtypeStruct((B, D), f32) for _ in range(2)],
        scratch_types=(
            [pltpu.VMEM((_CH,), jnp.int32) for _ in range(4)]
            + [pltpu.VMEM((_CH, D), f32) for _ in range(4)]
            + [pltpu.SemaphoreType.DMA for _ in range(4)]
        ),
    )
    def gather_k(uidx_h, iidx_h, t0_h, t1_h,
                 o0_h, o1_h,
                 uv0, uv1, iv0, iv1,
                 b00, b10, b01, b11,
                 g0, g1, s0, s1):
        uv = (uv0, uv1)
        iv = (iv0, iv1)
        ubuf = (b00, b01)
        ibuf = (b10, b11)
        gsem = (g0, g1)
        ssem = (s0, s1)
        wid = lax.axis_index("s") * _NC + lax.axis_index("c")
        base = wid * _BPW

        jobs = [(t0_h, t1_h, o0_h, o1_h, c) for c in range(_NCH)]
        nj = len(jobs)

        gh = [None] * nj
        sh = [None] * nj
        pltpu.sync_copy(uidx_h.at[pl.ds(base + jobs[0][4] * _CH, _CH)], uv[0])
        pltpu.sync_copy(iidx_h.at[pl.ds(base + jobs[0][4] * _CH, _CH)], iv[0])
        for j in range(nj):
            p = j % 2
            tu, ti, ou, oi, c = jobs[j]
            if j >= 2:
                sh[j - 2][0].wait()
                sh[j - 2][1].wait()
            gh[j] = (pltpu.async_copy(tu.at[uv[p]], ubuf[p], gsem[p]),
                     pltpu.async_copy(ti.at[iv[p]], ibuf[p], gsem[p]))
            if j >= 1:
                q = 1 - p
                tu_p, ti_p, ou_p, oi_p, c_p = jobs[j - 1]
                off_p = base + c_p * _CH
                gh[j - 1][0].wait()
                gh[j - 1][1].wait()
                sh[j - 1] = (
                    pltpu.async_copy(ubuf[q], ou_p.at[pl.ds(off_p, _CH)],
                                     ssem[q]),
                    pltpu.async_copy(ibuf[q], oi_p.at[pl.ds(off_p, _CH)],
                                     ssem[q]),
                )
            if j + 1 < nj:
                off_n = base + jobs[j + 1][4] * _CH
                pltpu.sync_copy(uidx_h.at[pl.ds(off_n, _CH)], uv[1 - p])
                pltpu.sync_copy(iidx_h.at[pl.ds(off_n, _CH)], iv[1 - p])
        j = nj - 1
        p = j % 2
        gh[j][0].wait()
        gh[j][1].wait()
        tu_p, ti_p, ou_p, oi_p, c_p = jobs[j]
        off_p = base + c_p * _CH
        sh[j] = (pltpu.async_copy(ubuf[p], ou_p.at[pl.ds(off_p, _CH)],
                                  ssem[p]),
                 pltpu.async_copy(ibuf[p], oi_p.at[pl.ds(off_p, _CH)],
                                  ssem[p]))
        sh[j - 1][0].wait()
        sh[j - 1][1].wait()
        sh[j][0].wait()
        sh[j][1].wait()

    return gather_k(uidx, iidx, t0, t1)


_BLK = 2048
_NB = B // _BLK


def _leaky(z):
    return jnp.where(z > 0, z, 0.1 * z)


def _accum_stats(a, st_ref):
    ps = jnp.stack([jnp.sum(a, axis=0), jnp.sum(a * a, axis=0)])

    @pl.when(pl.program_id(0) == 0)
    def _():
        st_ref[...] = ps

    @pl.when(pl.program_id(0) > 0)
    def _():
        st_ref[...] = st_ref[...] + ps


def _norm_params(st, g, be):
    m = st[0] * (1.0 / B)
    var = st[1] * (1.0 / B) - m * m
    scale = g * lax.rsqrt(var + EPS)
    shift = be - m * scale
    return scale, shift


def _accum_scratch(a, st_ref):
    ps = jnp.stack([jnp.sum(a, axis=0), jnp.sum(a * a, axis=0)])

    @pl.when(pl.program_id(1) == 0)
    def _():
        st_ref[...] = ps

    @pl.when(pl.program_id(1) > 0)
    def _():
        st_ref[...] = st_ref[...] + ps


def _mega_body(um_ref, im_ref, ug_ref, ig_ref,
               w1_ref, b1_ref, g1_ref, be1_ref,
               w2_ref, b2_ref, g2_ref, be2_ref,
               w3_ref, b3_ref, g3_ref, be3_ref,
               wo_ref, bo_ref,
               o_ref,
               h1s, h2s, h3s, st1, st2, st3):
    s = pl.program_id(0)
    i = pl.program_id(1)
    rows = pl.ds(i * _BLK, _BLK)
    bf16 = jnp.bfloat16
    f32 = jnp.float32

    @pl.when(s == 0)
    def _():
        w = w1_ref[...]
        z = (jnp.dot(um_ref[...].astype(bf16), w[:D],
                     preferred_element_type=f32)
             + jnp.dot(im_ref[...].astype(bf16), w[D:],
                       preferred_element_type=f32)
             + b1_ref[...])
        a = _leaky(z)
        h1s[rows, :] = a.astype(bf16)
        _accum_scratch(a, st1)

    @pl.when(s == 1)
    def _():
        scale, shift = _norm_params(st1[...], g1_ref[...], be1_ref[...])
        x = (h1s[rows, :].astype(f32) * scale + shift).astype(bf16)
        z = (jnp.dot(x, w2_ref[...], preferred_element_type=f32)
             + b2_ref[...])
        a = _leaky(z)
        h2s[rows, :] = a.astype(bf16)
        _accum_scratch(a, st2)

    @pl.when(s == 2)
    def _():
        scale, shift = _norm_params(st2[...], g2_ref[...], be2_ref[...])
        x = (h2s[rows, :].astype(f32) * scale + shift).astype(bf16)
        z = (jnp.dot(x, w3_ref[...], preferred_element_type=f32)
             + b3_ref[...])
        a = _leaky(z)
        h3s[rows, :] = a.astype(bf16)
        _accum_scratch(a, st3)

    @pl.when(s == 3)
    def _():
        scale, shift = _norm_params(st3[...], g3_ref[...], be3_ref[...])
        z3 = h3s[rows, :].astype(f32) * scale + shift
        gmf = ug_ref[...] * ig_ref[...]
        wo = wo_ref[...][:, 0]
        r = jnp.sum(gmf * wo[:D] + z3 * wo[D:], axis=1) + bo_ref[0]
        o_ref[...] = jax.nn.sigmoid(r)


def kernel(user_indices, item_indices, ue_gmf, ie_gmf, ue_mlp, ie_mlp,
           W1, b1, g1, be1, W2, b2, g2, be2, W3, b3, g3, be3, Wo, bo):
    uidx = user_indices.astype(jnp.int32)
    iidx = item_indices.astype(jnp.int32)

    um, im = _sc_gather2(uidx, iidx, ue_mlp, ie_mlp)
    ug, ig = _sc_gather2(uidx, iidx, ue_gmf, ie_gmf)

    f32 = jnp.float32
    bf16 = jnp.bfloat16

    def stage0_rows(h):
        return pl.BlockSpec(
            (_BLK, h), lambda s, i: (jnp.where(s == 0, i, 0), 0))

    def stage3_rows(h):
        return pl.BlockSpec(
            (_BLK, h), lambda s, i: (jnp.where(s == 3, i, 0), 0))

    def const2():
        return pl.BlockSpec(None, lambda s, i: (0, 0))

    def const1():
        return pl.BlockSpec(None, lambda s, i: (0,))

    out = pl.pallas_call(
        _mega_body,
        grid=(4, _NB),
        in_specs=[stage0_rows(D), stage0_rows(D),
                  stage3_rows(D), stage3_rows(D),
                  const2(), const1(), const1(), const1(),
                  const2(), const1(), const1(), const1(),
                  const2(), const1(), const1(), const1(),
                  const2(), const1()],
        out_specs=pl.BlockSpec((_BLK,), lambda s, i: (jnp.where(s == 3, i, 0),)),
        out_shape=jax.ShapeDtypeStruct((B,), f32),
        scratch_shapes=[
            pltpu.VMEM((B, 512), bf16),
            pltpu.VMEM((B, 256), bf16),
            pltpu.VMEM((B, 128), bf16),
            pltpu.VMEM((2, 512), f32),
            pltpu.VMEM((2, 256), f32),
            pltpu.VMEM((2, 128), f32),
        ],
    )(um, im, ug, ig,
      W1.astype(bf16), b1, g1, be1,
      W2.astype(bf16), b2, g2, be2,
      W3.astype(bf16), b3, g3, be3,
      Wo, bo)

    return out
